# Initial kernel scaffold; baseline (speedup 1.0000x reference)
import jax
import jax.numpy as jnp
from jax.experimental import pallas as pl


def kernel(edge_index, Gu, Gi):
    raise NotImplementedError("write your pallas kernel here")


# speedup vs baseline: 11.6275x; 11.6275x over previous
import functools

import jax
import jax.numpy as jnp
from jax import lax
from jax.experimental import pallas as pl
from jax.experimental.pallas import tpu as pltpu
from jax.experimental.pallas import tpu_sc as plsc

N_USERS = 20000
N_ITEMS = 30000
N_NODES = N_USERS + N_ITEMS
K = 64
N_EDGES_DIR = 1600000

NC = 2
NS = 16
HALF = N_NODES // NC
ACC_ROWS = 25088
RO_CH = 1568
DUMMY0 = 25024

U = 128
UNITS_TOTAL = 12544
U_PER_TILE = UNITS_TOTAL // NS
U_CH = 28
N_OCH = U_PER_TILE // U_CH
RO_SUB = 98

DEG_W = 16


def _rebase_unit(dst_v, k, base):
    for c in range(U // 16):
        d = dst_v[k, pl.ds(c * 16, 16)]
        inr = (d >= base) & (d < base + HALF)
        dummy = DUMMY0 + (c % 4) * 16 + lax.iota(jnp.int32, 16)
        dst_v[k, pl.ds(c * 16, 16)] = jnp.where(inr, d - base, dummy)


def _zero_vmem(buf, rows, width):
    z = jnp.zeros((16,), jnp.float32)

    def body(r, _):
        for c in range(width // 16):
            buf[r, pl.ds(c * 16, 16)] = z
        return 0

    lax.fori_loop(0, rows, body, 0)


def _sc_mesh():
    return plsc.VectorSubcoreMesh(core_axis_name="c", subcore_axis_name="s")


_SC_PARAMS = pltpu.CompilerParams(use_tc_tiling_on_sc=False)


def _deg_body(dst_hbm, out_hbm, dst_v, ones_v, stg, acc, sem):
    h = lax.axis_index("c")
    s = lax.axis_index("s")
    base = h * HALF

    one = jnp.full((16,), 1.0, jnp.float32)

    def fill(r, _):
        ones_v[r, pl.ds(0, 16)] = one
        return 0

    lax.fori_loop(0, U, fill, 0)

    _zero_vmem(stg, RO_CH, DEG_W)
    pltpu.sync_copy(stg, acc.at[pl.ds(s * RO_CH, RO_CH)])
    plsc.subcore_barrier()

    def outer(oc, _):
        u0 = s * U_PER_TILE + oc * U_CH
        pltpu.sync_copy(dst_hbm.at[pl.ds(u0, U_CH)], dst_v)

        def inner(k, _):
            _rebase_unit(dst_v, k, base)
            pltpu.sync_copy(ones_v, acc.at[dst_v.at[k]], add=True)
            return 0

        lax.fori_loop(0, U_CH, inner, 0)
        return 0

    lax.fori_loop(0, N_OCH, outer, 0)
    plsc.subcore_barrier()

    pltpu.sync_copy(acc.at[pl.ds(s * RO_CH, RO_CH)], stg)
    pltpu.sync_copy(stg, out_hbm.at[h, pl.ds(s * RO_CH, RO_CH)])


_deg_call = pl.kernel(
    _deg_body,
    out_type=jax.ShapeDtypeStruct((NC, ACC_ROWS, DEG_W), jnp.float32),
    mesh=_sc_mesh(),
    scratch_types=[
        pltpu.VMEM((U_CH, U), jnp.int32),
        pltpu.VMEM((U, DEG_W), jnp.float32),
        pltpu.VMEM((RO_CH, DEG_W), jnp.float32),
        pltpu.VMEM_SHARED((ACC_ROWS, DEG_W), jnp.float32),
        pltpu.SemaphoreType.DMA,
    ],
    compiler_params=_SC_PARAMS,
)


def _spmm_body(src_hbm, dst_hbm, xh_hbm, z_hbm, src_v, dst_v, rows_v, stg,
               acc, sem):
    h = lax.axis_index("c")
    s = lax.axis_index("s")
    base = h * HALF

    _zero_vmem(stg, RO_SUB, K)

    def zcp(i, _):
        pltpu.sync_copy(stg, acc.at[pl.ds(s * RO_CH + i * RO_SUB, RO_SUB)])
        return 0

    lax.fori_loop(0, RO_CH // RO_SUB, zcp, 0)
    plsc.subcore_barrier()

    def outer(oc, _):
        u0 = s * U_PER_TILE + oc * U_CH
        pltpu.sync_copy(src_hbm.at[pl.ds(u0, U_CH)], src_v)
        pltpu.sync_copy(dst_hbm.at[pl.ds(u0, U_CH)], dst_v)

        def inner(k, _):
            _rebase_unit(dst_v, k, base)
            pltpu.async_copy(xh_hbm.at[src_v.at[k]], rows_v, sem).wait()
            pltpu.sync_copy(rows_v, acc.at[dst_v.at[k]], add=True)
            return 0

        lax.fori_loop(0, U_CH, inner, 0)
        return 0

    lax.fori_loop(0, N_OCH, outer, 0)
    plsc.subcore_barrier()

    def rocp(i, _):
        r0 = s * RO_CH + i * RO_SUB
        pltpu.sync_copy(acc.at[pl.ds(r0, RO_SUB)], stg)
        pltpu.sync_copy(stg, z_hbm.at[h, pl.ds(r0, RO_SUB)])
        return 0

    lax.fori_loop(0, RO_CH // RO_SUB, rocp, 0)


_spmm_call = pl.kernel(
    _spmm_body,
    out_type=jax.ShapeDtypeStruct((NC, ACC_ROWS, K), jnp.float32),
    mesh=_sc_mesh(),
    scratch_types=[
        pltpu.VMEM((U_CH, U), jnp.int32),
        pltpu.VMEM((U_CH, U), jnp.int32),
        pltpu.VMEM((U, K), jnp.float32),
        pltpu.VMEM((RO_SUB, K), jnp.float32),
        pltpu.VMEM_SHARED((ACC_ROWS, K), jnp.float32),
        pltpu.SemaphoreType.DMA,
    ],
    compiler_params=_SC_PARAMS,
)


TC_B = 200
TC_GRID = (NC, HALF // TC_B)


def _deg_spec():
    return pl.BlockSpec((1, TC_B, DEG_W), lambda h, i: (h, i, 0))


def _emb_spec():
    return pl.BlockSpec((1, TC_B, K), lambda h, i: (h, i, 0))


def _prep_body(deg_ref, ego_ref, xh_ref):
    d = jnp.maximum(deg_ref[0, :, 0:1], 1.0)
    xh_ref[0] = ego_ref[0] * lax.rsqrt(d)


def _scale_body(deg_ref, z_ref, xh_ref):
    d = jnp.maximum(deg_ref[0, :, 0:1], 1.0)
    xh_ref[0] = z_ref[0] / d


def _final_body(deg_ref, ego_ref, z1_ref, z2_ref, z3_ref, out_ref):
    d = jnp.maximum(deg_ref[0, :, 0:1], 1.0)
    zs = z1_ref[0] + z2_ref[0] + z3_ref[0]
    out_ref[0] = (ego_ref[0] + zs * lax.rsqrt(d)) * 0.25


def _tc_call(body, n_emb_inputs):
    return pl.pallas_call(
        body,
        grid=TC_GRID,
        in_specs=[_deg_spec()] + [_emb_spec()] * n_emb_inputs,
        out_specs=_emb_spec(),
        out_shape=jax.ShapeDtypeStruct((NC, HALF, K), jnp.float32),
    )


def kernel(edge_index, Gu, Gi):
    e0 = edge_index[0].astype(jnp.int32)
    e1 = edge_index[1].astype(jnp.int32)
    npad = UNITS_TOTAL * U - 2 * (e0.shape[0])
    pad_src = (jnp.arange(npad, dtype=jnp.int32) * 37) % N_NODES
    pad_dst = jnp.full((npad,), -1, jnp.int32)
    src2d = jnp.concatenate([e0, e1, pad_src]).reshape(UNITS_TOTAL, U)
    dst2d = jnp.concatenate([e1, e0, pad_dst]).reshape(UNITS_TOTAL, U)
    ego = jnp.concatenate([Gu, Gi], axis=0)
    ego3d = ego.reshape(NC, HALF, K)

    deg = _deg_call(dst2d)

    xh0 = _tc_call(_prep_body, 1)(deg, ego3d).reshape(N_NODES, K)
    z1 = _spmm_call(src2d, dst2d, xh0)
    xh1 = _tc_call(_scale_body, 1)(deg, z1).reshape(N_NODES, K)
    z2 = _spmm_call(src2d, dst2d, xh1)
    xh2 = _tc_call(_scale_body, 1)(deg, z2).reshape(N_NODES, K)
    z3 = _spmm_call(src2d, dst2d, xh2)
    out = _tc_call(_final_body, 4)(deg, ego3d, z1, z2, z3)
    out = out.reshape(N_NODES, K)
    return out[:N_USERS], out[N_USERS:]


# speedup vs baseline: 29.2590x; 2.5164x over previous
import jax
import jax.numpy as jnp
from jax import lax
from jax.experimental import pallas as pl
from jax.experimental.pallas import tpu as pltpu
from jax.experimental.pallas import tpu_sc as plsc

N_USERS = 20000
N_ITEMS = 30000
N_NODES = N_USERS + N_ITEMS
K = 64
KH = 32

NC = 2
NS = 16

U = 128
UNITS = 6400
U_PER_TILE = UNITS // NS
U_CH = 20
N_CH = U_PER_TILE // U_CH
NB = 4
GRP = (2 * U_CH) // NB

ACC_ROWS = 50176
ZCH = 112
RO_CH = 125
TBL_ROWS = 50176

HALF = N_NODES // NC
DEG_ROWS = 25088
DEG_RO = 1568
DUMMY0 = 25024
DEG_W = 16


def _sc_mesh():
    return plsc.VectorSubcoreMesh(core_axis_name="c", subcore_axis_name="s")


_SC_PARAMS = pltpu.CompilerParams(use_tc_tiling_on_sc=False)


def _deg_body(ep_hbm, out_hbm, dst_v, ones_v, stg, acc, sem):
    h = lax.axis_index("c")
    s = lax.axis_index("s")
    base = h * HALF

    one = jnp.full((16,), 1.0, jnp.float32)

    def fill(r, _):
        ones_v[r, pl.ds(0, 16)] = one
        return 0

    lax.fori_loop(0, U, fill, 0)

    zero = jnp.zeros((16,), jnp.float32)

    def zr(r, _):
        stg[r, pl.ds(0, 16)] = zero
        return 0

    lax.fori_loop(0, DEG_RO, zr, 0)
    pltpu.sync_copy(stg, acc.at[pl.ds(s * DEG_RO, DEG_RO)])
    plsc.subcore_barrier()

    def outer(oc, _):
        u0 = s * U_PER_TILE + oc * U_CH
        for d in range(2):
            pltpu.sync_copy(ep_hbm.at[1 - d, pl.ds(u0, U_CH)], dst_v)

            def inner(k, _):
                for c in range(U // 16):
                    dd = dst_v[k, pl.ds(c * 16, 16)]
                    inr = (dd >= base) & (dd < base + HALF)
                    dum = DUMMY0 + (c % 4) * 16 + lax.iota(jnp.int32, 16)
                    dst_v[k, pl.ds(c * 16, 16)] = jnp.where(inr, dd - base, dum)
                pltpu.sync_copy(ones_v, acc.at[dst_v.at[k]], add=True)
                return 0

            lax.fori_loop(0, U_CH, inner, 0)
        return 0

    lax.fori_loop(0, N_CH, outer, 0)
    plsc.subcore_barrier()

    pltpu.sync_copy(acc.at[pl.ds(s * DEG_RO, DEG_RO)], stg)
    pltpu.sync_copy(stg, out_hbm.at[h, pl.ds(s * DEG_RO, DEG_RO)])


_deg_call = pl.kernel(
    _deg_body,
    out_type=jax.ShapeDtypeStruct((NC, DEG_ROWS, DEG_W), jnp.float32),
    mesh=_sc_mesh(),
    scratch_types=[
        pltpu.VMEM((U_CH, U), jnp.int32),
        pltpu.VMEM((U, DEG_W), jnp.float32),
        pltpu.VMEM((DEG_RO, DEG_W), jnp.float32),
        pltpu.VMEM_SHARED((DEG_ROWS, DEG_W), jnp.float32),
        pltpu.SemaphoreType.DMA,
    ],
    compiler_params=_SC_PARAMS,
)


def _mega_body(ep_hbm, xh0_hbm, dinv_hbm, xh1_hbm, xh2_hbm, z3_hbm,
               e0_v, e1_v, r0_v, r1_v, r2_v, r3_v, acc_ref,
               sem0, sem1, sem2, sem3):
    h = lax.axis_index("c")
    s = lax.axis_index("s")
    rows = (r0_v, r1_v, r2_v, r3_v)
    sems = (sem0, sem1, sem2, sem3)

    def zero_r3():
        zero = jnp.zeros((16,), jnp.float32)

        def zr(r, _):
            for c in range(KH // 16):
                r3_v[r, pl.ds(c * 16, 16)] = zero
            return 0

        lax.fori_loop(0, U, zr, 0)

    def body_with_acc(acc):
        zero_r3()

        def zcp(i, _):
            pltpu.sync_copy(r3_v.at[pl.ds(0, ZCH)],
                            acc.at[pl.ds(s * (ACC_ROWS // NS) + i * ZCH, ZCH)])
            return 0

        lax.fori_loop(0, ACC_ROWS // (NS * ZCH), zcp, 0)
        plsc.subcore_barrier()

        def scatter_phase(table_hbm):
            def chunk(oc, _):
                u0 = s * U_PER_TILE + oc * U_CH
                pltpu.sync_copy(ep_hbm.at[0, pl.ds(u0, U_CH)], e0_v)
                pltpu.sync_copy(ep_hbm.at[1, pl.ds(u0, U_CH)], e1_v)

                def fire(g, j):
                    k = 2 * g + (j // 2)
                    gsrc = e0_v if j % 2 == 0 else e1_v
                    pltpu.async_copy(table_hbm.at[gsrc.at[k]], rows[j],
                                     sems[j])

                def scat(g, j):
                    k = 2 * g + (j // 2)
                    gdst = e1_v if j % 2 == 0 else e0_v
                    pltpu.sync_copy(rows[j], acc.at[gdst.at[k]], add=True)

                for j in range(NB):
                    fire(0, j)

                def grp(g, _):
                    for j in range(NB):
                        pltpu.make_async_copy(
                            table_hbm.at[pl.ds(0, U)], rows[j], sems[j]
                        ).wait()
                        scat(g, j)

                        @pl.when(g < GRP - 1)
                        def _():
                            fire(g + 1, j)

                    return 0

                lax.fori_loop(0, GRP, grp, 0)
                return 0

            lax.fori_loop(0, N_CH, chunk, 0)
            plsc.subcore_barrier()

        def readout_phase(out_hbm, scale, rezero):
            if rezero:
                zero_r3()

            def ro(i, _):
                r0 = s * (N_NODES // NS) + i * RO_CH
                pltpu.sync_copy(acc.at[pl.ds(r0, RO_CH)],
                                r0_v.at[pl.ds(0, RO_CH)])
                if scale:
                    pltpu.sync_copy(dinv_hbm.at[pl.ds(r0, RO_CH)],
                                    r1_v.at[pl.ds(0, RO_CH)])

                    def mrow(r, _):
                        for c in range(KH // 16):
                            sl = pl.ds(c * 16, 16)
                            r0_v[r, sl] = r0_v[r, sl] * r1_v[r, sl]
                        return 0

                    lax.fori_loop(0, RO_CH, mrow, 0)
                pltpu.sync_copy(r0_v.at[pl.ds(0, RO_CH)],
                                out_hbm.at[h, pl.ds(r0, RO_CH)])
                if rezero:
                    pltpu.sync_copy(r3_v.at[pl.ds(0, RO_CH)],
                                    acc.at[pl.ds(r0, RO_CH)])
                return 0

            lax.fori_loop(0, N_NODES // (NS * RO_CH), ro, 0)
            plsc.subcore_barrier()

        scatter_phase(xh0_hbm.at[h])
        readout_phase(xh1_hbm, scale=True, rezero=True)
        scatter_phase(xh1_hbm.at[h])
        readout_phase(xh2_hbm, scale=True, rezero=True)
        scatter_phase(xh2_hbm.at[h])
        readout_phase(z3_hbm, scale=False, rezero=False)

    body_with_acc(acc_ref)


_mega_call = pl.kernel(
    _mega_body,
    out_type=(
        jax.ShapeDtypeStruct((NC, TBL_ROWS, KH), jnp.float32),
        jax.ShapeDtypeStruct((NC, TBL_ROWS, KH), jnp.float32),
        jax.ShapeDtypeStruct((NC, TBL_ROWS, KH), jnp.float32),
    ),
    mesh=_sc_mesh(),
    scratch_types=[
        pltpu.VMEM((U_CH, U), jnp.int32),
        pltpu.VMEM((U_CH, U), jnp.int32),
        pltpu.VMEM((U, KH), jnp.float32),
        pltpu.VMEM((U, KH), jnp.float32),
        pltpu.VMEM((U, KH), jnp.float32),
        pltpu.VMEM((U, KH), jnp.float32),
        pltpu.VMEM_SHARED((ACC_ROWS, KH), jnp.float32),
        pltpu.SemaphoreType.DMA,
        pltpu.SemaphoreType.DMA,
        pltpu.SemaphoreType.DMA,
        pltpu.SemaphoreType.DMA,
    ],
    compiler_params=_SC_PARAMS,
)


TC_B = 200
TC_GRID = (NC, HALF // TC_B)


def _deg_spec():
    return pl.BlockSpec((1, TC_B, DEG_W), lambda h, i: (h, i, 0))


def _ego_spec():
    return pl.BlockSpec((1, TC_B, K), lambda h, i: (h, i, 0))


def _col_spec():
    return pl.BlockSpec((NC, TC_B, KH), lambda h, i: (0, h * (HALF // TC_B) + i, 0))


def _flat_spec():
    return pl.BlockSpec((TC_B, KH), lambda h, i: (h * (HALF // TC_B) + i, 0))


def _prep_body(deg_ref, ego_ref, xh0_ref, dinv_ref):
    d = jnp.maximum(deg_ref[0, :, 0:1], 1.0)
    rs = lax.rsqrt(d)
    xh0_ref[0] = ego_ref[0, :, :KH] * rs
    xh0_ref[1] = ego_ref[0, :, KH:] * rs
    dinv_ref[...] = jnp.broadcast_to(1.0 / d, (TC_B, KH))


def _final_body(deg_ref, ego_ref, xh1_ref, xh2_ref, z3_ref, out_ref):
    d = jnp.maximum(deg_ref[0, :, 0:1], 1.0)
    rs = lax.rsqrt(d)
    for p in range(NC):
        zs = (xh1_ref[p] + xh2_ref[p]) * d + z3_ref[p]
        out_ref[0, :, p * KH:(p + 1) * KH] = (
            ego_ref[0, :, p * KH:(p + 1) * KH] + zs * rs) * 0.25


_prep_call = pl.pallas_call(
    _prep_body,
    grid=TC_GRID,
    in_specs=[_deg_spec(), _ego_spec()],
    out_specs=(_col_spec(), _flat_spec()),
    out_shape=(
        jax.ShapeDtypeStruct((NC, TBL_ROWS, KH), jnp.float32),
        jax.ShapeDtypeStruct((N_NODES, KH), jnp.float32),
    ),
)

_final_call = pl.pallas_call(
    _final_body,
    grid=TC_GRID,
    in_specs=[_deg_spec(), _ego_spec(), _col_spec(), _col_spec(), _col_spec()],
    out_specs=_ego_spec(),
    out_shape=jax.ShapeDtypeStruct((NC, HALF, K), jnp.float32),
)


def kernel(edge_index, Gu, Gi):
    e0 = edge_index[0].astype(jnp.int32)
    e1 = edge_index[1].astype(jnp.int32)
    npad = UNITS * U - e0.shape[0]
    ar = jnp.arange(npad, dtype=jnp.int32)
    pe0 = jnp.concatenate([e0, N_NODES + (ar % 64)])
    pe1 = jnp.concatenate([e1, N_NODES + 64 + (ar % 64)])
    ep = jnp.stack([pe0, pe1]).reshape(2, UNITS, U)
    ego = jnp.concatenate([Gu, Gi], axis=0)
    ego3d = ego.reshape(NC, HALF, K)

    deg = _deg_call(ep)
    xh0, dinv = _prep_call(deg, ego3d)
    xh1, xh2, z3 = _mega_call(ep, xh0, dinv)
    out = _final_call(deg, ego3d, xh1, xh2, z3)
    out = out.reshape(N_NODES, K)
    return out[:N_USERS], out[N_USERS:]
